# baseline (speedup 1.0000x reference)
import functools

import jax
import jax.numpy as jnp
from jax import lax
from jax.experimental import pallas as pl
from jax.experimental.pallas import tpu as pltpu
from jax.experimental.pallas import tpu_sc as plsc

N = 10000
E = 160000
C = 256
CH = 128

NC = 2
NS = 16
NW = NC * NS

EPT1 = 5000
DEGN = 10240

CHUNK = 128
WCH = 40
NWIN = 2
EPT2 = NWIN * WCH * CHUNK
ROWS_PER_TILE = 640
ACC_ROWS = ROWS_PER_TILE * NS
COL_DUMMY = 10008

BLK = 1000
GRID = N // BLK
BLK2 = 2000
GRID2 = N // BLK2


def _sc_mesh():
  return plsc.VectorSubcoreMesh(core_axis_name="c", subcore_axis_name="s",
                                num_cores=NC, num_subcores=NS)


def _sc_degree_body(row_hbm, out_hbm, idx_v, deg_v):
  c = lax.axis_index("c")
  s = lax.axis_index("s")
  wid = c * NS + s
  pltpu.sync_copy(row_hbm.at[pl.ds(wid * EPT1, EPT1)], idx_v)

  def zero_body(i, carry):
    deg_v[pl.ds(i * 16, 16)] = jnp.zeros((16,), jnp.float32)
    return carry
  lax.fori_loop(0, DEGN // 16, zero_body, 0)

  ones16 = jnp.ones((16,), jnp.float32)

  def scat_body(i, carry):
    idxs = idx_v[pl.ds(i * 16, 16)]
    plsc.addupdate_scatter(deg_v, [idxs], ones16)
    return carry
  lax.fori_loop(0, EPT1 // 16, scat_body, 0)

  tail = EPT1 % 16
  if tail:
    skip = 16 - tail
    idxs = idx_v[pl.ds(EPT1 - 16, 16)]
    mask = lax.iota(jnp.int32, 16) >= skip
    idxs = jnp.where(mask, idxs, N)
    plsc.addupdate_scatter(deg_v, [idxs], ones16, mask=mask)

  pltpu.sync_copy(deg_v, out_hbm.at[wid])


def _sc_degree(row1):
  return pl.kernel(
      _sc_degree_body,
      out_type=jax.ShapeDtypeStruct((NW, DEGN), jnp.float32),
      mesh=_sc_mesh(),
      scratch_types=[
          pltpu.VMEM((EPT1,), jnp.int32),
          pltpu.VMEM((DEGN,), jnp.float32),
      ],
      compiler_params=pltpu.CompilerParams(needs_layout_passes=False),
  )(row1)


def _sc_edge_agg_body(g_hbm, row_hbm, col_hbm, zero_hbm, out_hbm,
                      roww, colw, buf0, buf1, acc, semA, semB, semC, semD):
  c = lax.axis_index("c")
  s = lax.axis_index("s")
  wid = c * NS + s

  pltpu.sync_copy(zero_hbm, buf0)
  base = s * ROWS_PER_TILE
  nfull = ROWS_PER_TILE // CHUNK
  rem = ROWS_PER_TILE % CHUNK
  for k in range(nfull):
    pltpu.sync_copy(buf0, acc.at[pl.ds(base + k * CHUNK, CHUNK)])
  if rem:
    pltpu.sync_copy(buf0.at[pl.ds(0, rem)],
                    acc.at[pl.ds(base + nfull * CHUNK, rem)])
  plsc.subcore_barrier()

  def window(w, carry):
    pltpu.sync_copy(row_hbm.at[wid * NWIN + w], roww)
    pltpu.sync_copy(col_hbm.at[wid * NWIN + w], colw)
    pltpu.async_copy(g_hbm.at[roww.at[0]], buf0, semA)
    pltpu.async_copy(g_hbm.at[roww.at[1]], buf1, semB)

    def pair(k, carry2):
      p = 2 * k
      pltpu.make_async_copy(g_hbm.at[roww.at[p]], buf0, semA).wait()
      pltpu.async_copy(buf0, acc.at[colw.at[p]], semC, add=True)
      pltpu.make_async_copy(g_hbm.at[roww.at[p + 1]], buf1, semB).wait()
      pltpu.async_copy(buf1, acc.at[colw.at[p + 1]], semD, add=True)
      pltpu.make_async_copy(buf0, acc.at[colw.at[p]], semC).wait()
      pltpu.async_copy(g_hbm.at[roww.at[p + 2]], buf0, semA)
      pltpu.make_async_copy(buf1, acc.at[colw.at[p + 1]], semD).wait()
      pltpu.async_copy(g_hbm.at[roww.at[p + 3]], buf1, semB)
      return carry2
    lax.fori_loop(0, WCH // 2 - 1, pair, 0)

    last = WCH - 2
    pltpu.make_async_copy(g_hbm.at[roww.at[last]], buf0, semA).wait()
    pltpu.async_copy(buf0, acc.at[colw.at[last]], semC, add=True)
    pltpu.make_async_copy(g_hbm.at[roww.at[last + 1]], buf1, semB).wait()
    pltpu.async_copy(buf1, acc.at[colw.at[last + 1]], semD, add=True)
    pltpu.make_async_copy(buf0, acc.at[colw.at[last]], semC).wait()
    pltpu.make_async_copy(buf1, acc.at[colw.at[last + 1]], semD).wait()
    return carry
  lax.fori_loop(0, NWIN, window, 0)

  plsc.subcore_barrier()
  pltpu.sync_copy(acc.at[pl.ds(base, ROWS_PER_TILE)],
                  out_hbm.at[pl.ds(c * ACC_ROWS + base, ROWS_PER_TILE)])


def _sc_edge_agg(g_flat, row2, col2, zeros128):
  return pl.kernel(
      _sc_edge_agg_body,
      out_type=jax.ShapeDtypeStruct((NC * ACC_ROWS, CH), jnp.float32),
      mesh=_sc_mesh(),
      scratch_types=[
          pltpu.VMEM((WCH, CHUNK), jnp.int32),
          pltpu.VMEM((WCH, CHUNK), jnp.int32),
          pltpu.VMEM((CHUNK, CH), jnp.float32),
          pltpu.VMEM((CHUNK, CH), jnp.float32),
          pltpu.VMEM_SHARED((ACC_ROWS, CH), jnp.float32),
          pltpu.SemaphoreType.DMA,
          pltpu.SemaphoreType.DMA,
          pltpu.SemaphoreType.DMA,
          pltpu.SemaphoreType.DMA,
      ],
      compiler_params=pltpu.CompilerParams(needs_layout_passes=False),
  )(g_flat, row2, col2, zeros128)


def _full(shape):
  return pl.BlockSpec(shape, lambda i: (0,) * len(shape))


def _tca1_body(x_ref, w1a_ref, b1a_ref, w1b_ref, b1b_ref,
               w1c_ref, b1c_ref, h1_ref):
  x = x_ref[...]
  p = x @ w1a_ref[...] + b1a_ref[...]
  q = jax.nn.sigmoid(x @ w1b_ref[...] + b1b_ref[...])
  r = x @ w1c_ref[...] + b1c_ref[...]
  h1_ref[...] = jax.nn.relu(p * q + r)


def _tc_a1(x, w1a, b1a, w1b, b1b, w1c, b1c):
  return pl.pallas_call(
      _tca1_body,
      grid=(GRID,),
      in_specs=[
          pl.BlockSpec((BLK, C), lambda i: (i, 0)),
          _full((C, C)), _full((1, C)),
          _full((C, C)), _full((1, C)),
          _full((C, C)), _full((1, C)),
      ],
      out_specs=pl.BlockSpec((BLK, C), lambda i: (i, 0)),
      out_shape=jax.ShapeDtypeStruct((N, C), jnp.float32),
  )(x, w1a, b1a, w1b, b1b, w1c, b1c)


def _tca2_body(h1_ref, degs_ref, g2_ref, dis_ref):
  h1 = h1_ref[...]
  d = lax.dot_general(degs_ref[...], jnp.ones((NW, 1), jnp.float32),
                      (((1,), (0,)), ((), ())))
  dis = jnp.where(d > 0, lax.rsqrt(jnp.where(d > 0, d, 1.0)), 0.0)
  dis_ref[...] = dis
  g = dis * h1
  g2_ref[0, :, :] = g[:, :CH]
  g2_ref[1, :, :] = g[:, CH:]


def _tc_a2(h1, degs):
  return pl.pallas_call(
      _tca2_body,
      grid=(GRID2,),
      in_specs=[
          pl.BlockSpec((BLK2, C), lambda i: (i, 0)),
          pl.BlockSpec((BLK2, NW), lambda i: (i, 0)),
      ],
      out_specs=[
          pl.BlockSpec((NC, BLK2, CH), lambda i: (0, i, 0)),
          pl.BlockSpec((BLK2, 1), lambda i: (i, 0)),
      ],
      out_shape=[
          jax.ShapeDtypeStruct((NC, N, CH), jnp.float32),
          jax.ShapeDtypeStruct((N, 1), jnp.float32),
      ],
  )(h1, degs)


def _tcb1_body(h1_ref, th0_ref, cb_ref, u_ref):
  u_ref[...] = h1_ref[...] @ th0_ref[...] + cb_ref[...]


def _tc_b1(h1, th0, cb):
  return pl.pallas_call(
      _tcb1_body,
      grid=(GRID,),
      in_specs=[
          pl.BlockSpec((BLK, C), lambda i: (i, 0)),
          _full((C, C)), _full((1, C)),
      ],
      out_specs=pl.BlockSpec((BLK, C), lambda i: (i, 0)),
      out_shape=jax.ShapeDtypeStruct((N, C), jnp.float32),
  )(h1, th0, cb)


def _tcb2_body(u_ref, t2_ref, dis_ref, th1_ref,
               w2a_ref, b2a_ref, w2b_ref, b2b_ref, w2c_ref, b2c_ref,
               gamma_ref, beta_ref, wl_ref, bl_ref, out_ref):
  dis = dis_ref[...]
  t = jnp.concatenate([t2_ref[0, :, :], t2_ref[1, :, :]], axis=1)
  tx1 = (-dis) * t
  h2 = jax.nn.relu(u_ref[...] + tx1 @ th1_ref[...])

  p = h2 @ w2a_ref[...] + b2a_ref[...]
  q = jax.nn.sigmoid(h2 @ w2b_ref[...] + b2b_ref[...])
  r = h2 @ w2c_ref[...] + b2c_ref[...]
  h3 = jax.nn.relu(p * q + r)

  mean = jnp.mean(h3, axis=1, keepdims=True)
  var = jnp.mean((h3 - mean) ** 2, axis=1, keepdims=True)
  hn = (h3 - mean) / jnp.sqrt(var + 1e-5) * gamma_ref[...] + beta_ref[...]
  h4 = jax.nn.relu(hn)
  out_ref[...] = (jnp.sum(h4 * wl_ref[...], axis=1, keepdims=True)
                  + bl_ref[...])


def _tc_b2(u, t2, dis, th1, w2a, b2a, w2b, b2b, w2c, b2c,
           gamma, beta, wl, bl):
  return pl.pallas_call(
      _tcb2_body,
      grid=(GRID,),
      in_specs=[
          pl.BlockSpec((BLK, C), lambda i: (i, 0)),
          pl.BlockSpec((NC, BLK, CH), lambda i: (0, i, 0)),
          pl.BlockSpec((BLK, 1), lambda i: (i, 0)),
          _full((C, C)),
          _full((C, C)), _full((1, C)),
          _full((C, C)), _full((1, C)),
          _full((C, C)), _full((1, C)),
          pl.BlockSpec((BLK, 1), lambda i: (i, 0)),
          pl.BlockSpec((BLK, 1), lambda i: (i, 0)),
          _full((1, C)), _full((1, 1)),
      ],
      out_specs=pl.BlockSpec((BLK, 1), lambda i: (i, 0)),
      out_shape=jax.ShapeDtypeStruct((N, 1), jnp.float32),
  )(u, t2, dis, th1, w2a, b2a, w2b, b2b, w2c, b2c, gamma, beta, wl, bl)


def kernel(x, edge_index, W1a, b1a, W1b, b1b, W1c, b1c, th0, th1, cb,
           W2a, b2a, W2b, b2b, W2c, b2c, gamma, beta, Wl, bl):
  row = edge_index[0]
  col = edge_index[1]

  degs = _sc_degree(row)

  h1 = _tc_a1(x, W1a, b1a[None, :], W1b, b1b[None, :], W1c, b1c[None, :])
  g2, dis = _tc_a2(h1, degs.T)

  rowp = jnp.concatenate(
      [row, jnp.zeros((NS * EPT2 - E,), jnp.int32)]
  ).reshape(NS, NWIN, WCH, CHUNK)
  colp = jnp.concatenate(
      [col, jnp.full((NS * EPT2 - E,), COL_DUMMY, jnp.int32)]
  ).reshape(NS, NWIN, WCH, CHUNK)
  row2 = jnp.concatenate([rowp[None], rowp[None] + N], axis=0)
  row2 = row2.reshape(NW * NWIN, WCH, CHUNK)
  col2 = jnp.concatenate([colp[None], colp[None]], axis=0)
  col2 = col2.reshape(NW * NWIN, WCH, CHUNK)
  zeros128 = jnp.zeros((CHUNK, CH), jnp.float32)

  t_flat = _sc_edge_agg(g2.reshape(NC * N, CH), row2, col2, zeros128)
  t2 = t_flat.reshape(NC, ACC_ROWS, CH)

  u = _tc_b1(h1, th0, cb[None, :])
  return _tc_b2(u, t2, dis, th1,
                W2a, b2a[None, :], W2b, b2b[None, :], W2c, b2c[None, :],
                gamma[:, None], beta[:, None], Wl.reshape(1, C),
                bl.reshape(1, 1))


# speedup vs baseline: 1.0447x; 1.0447x over previous
import functools

import jax
import jax.numpy as jnp
from jax import lax
from jax.experimental import pallas as pl
from jax.experimental.pallas import tpu as pltpu
from jax.experimental.pallas import tpu_sc as plsc

N = 10000
E = 160000
C = 256
CH = 128

NC = 2
NS = 16
NW = NC * NS

EPT1 = 5008
DEGN = 10240
DEG_DUMMY = N

CHUNK = 128
WCH = 40
NWIN = 2
EPT2 = NWIN * WCH * CHUNK
ROWS_PER_TILE = 640
ACC_ROWS = ROWS_PER_TILE * NS
COL_DUMMY = 10008

BLK = 1000
GRID = N // BLK
BLK2 = 2000
GRID2 = N // BLK2


def _sc_mesh():
  return plsc.VectorSubcoreMesh(core_axis_name="c", subcore_axis_name="s",
                                num_cores=NC, num_subcores=NS)


def _sc_degree_body(row_hbm, out_hbm, idx_v, deg_v):
  c = lax.axis_index("c")
  s = lax.axis_index("s")
  wid = c * NS + s
  pltpu.sync_copy(row_hbm.at[wid], idx_v)

  def zero_body(i, carry):
    deg_v[pl.ds(i * 16, 16)] = jnp.zeros((16,), jnp.float32)
    return carry
  lax.fori_loop(0, DEGN // 16, zero_body, 0)

  ones16 = jnp.ones((16,), jnp.float32)

  def scat_body(i, carry):
    idxs = idx_v[pl.ds(i * 16, 16)]
    plsc.addupdate_scatter(deg_v, [idxs], ones16)
    return carry
  lax.fori_loop(0, EPT1 // 16, scat_body, 0)

  pltpu.sync_copy(deg_v, out_hbm.at[wid])


def _sc_degree(row1):
  return pl.kernel(
      _sc_degree_body,
      out_type=jax.ShapeDtypeStruct((NW, DEGN), jnp.float32),
      mesh=_sc_mesh(),
      scratch_types=[
          pltpu.VMEM((EPT1,), jnp.int32),
          pltpu.VMEM((DEGN,), jnp.float32),
      ],
      compiler_params=pltpu.CompilerParams(needs_layout_passes=False),
  )(row1)


def _sc_edge_agg_body(g_hbm, row_hbm, col_hbm, zero_hbm, out_hbm,
                      roww, colw, buf0, buf1, acc, semA, semB, semC, semD):
  c = lax.axis_index("c")
  s = lax.axis_index("s")
  wid = c * NS + s

  pltpu.sync_copy(zero_hbm, buf0)
  base = s * ROWS_PER_TILE
  nfull = ROWS_PER_TILE // CHUNK
  rem = ROWS_PER_TILE % CHUNK
  for k in range(nfull):
    pltpu.sync_copy(buf0, acc.at[pl.ds(base + k * CHUNK, CHUNK)])
  if rem:
    pltpu.sync_copy(buf0.at[pl.ds(0, rem)],
                    acc.at[pl.ds(base + nfull * CHUNK, rem)])
  plsc.subcore_barrier()

  def window(w, carry):
    pltpu.sync_copy(row_hbm.at[wid * NWIN + w], roww)
    pltpu.sync_copy(col_hbm.at[wid * NWIN + w], colw)
    pltpu.async_copy(g_hbm.at[roww.at[0]], buf0, semA)
    pltpu.async_copy(g_hbm.at[roww.at[1]], buf1, semB)

    def pair(k, carry2):
      p = 2 * k
      pltpu.make_async_copy(g_hbm.at[roww.at[p]], buf0, semA).wait()
      pltpu.async_copy(buf0, acc.at[colw.at[p]], semC, add=True)
      pltpu.make_async_copy(g_hbm.at[roww.at[p + 1]], buf1, semB).wait()
      pltpu.async_copy(buf1, acc.at[colw.at[p + 1]], semD, add=True)
      pltpu.make_async_copy(buf0, acc.at[colw.at[p]], semC).wait()
      pltpu.async_copy(g_hbm.at[roww.at[p + 2]], buf0, semA)
      pltpu.make_async_copy(buf1, acc.at[colw.at[p + 1]], semD).wait()
      pltpu.async_copy(g_hbm.at[roww.at[p + 3]], buf1, semB)
      return carry2
    lax.fori_loop(0, WCH // 2 - 1, pair, 0)

    last = WCH - 2
    pltpu.make_async_copy(g_hbm.at[roww.at[last]], buf0, semA).wait()
    pltpu.async_copy(buf0, acc.at[colw.at[last]], semC, add=True)
    pltpu.make_async_copy(g_hbm.at[roww.at[last + 1]], buf1, semB).wait()
    pltpu.async_copy(buf1, acc.at[colw.at[last + 1]], semD, add=True)
    pltpu.make_async_copy(buf0, acc.at[colw.at[last]], semC).wait()
    pltpu.make_async_copy(buf1, acc.at[colw.at[last + 1]], semD).wait()
    return carry
  lax.fori_loop(0, NWIN, window, 0)

  plsc.subcore_barrier()
  pltpu.sync_copy(acc.at[pl.ds(base, ROWS_PER_TILE)],
                  out_hbm.at[pl.ds(c * ACC_ROWS + base, ROWS_PER_TILE)])


def _sc_edge_agg(g_flat, row2, col2, zeros128):
  return pl.kernel(
      _sc_edge_agg_body,
      out_type=jax.ShapeDtypeStruct((NC * ACC_ROWS, CH), jnp.float32),
      mesh=_sc_mesh(),
      scratch_types=[
          pltpu.VMEM((WCH, CHUNK), jnp.int32),
          pltpu.VMEM((WCH, CHUNK), jnp.int32),
          pltpu.VMEM((CHUNK, CH), jnp.float32),
          pltpu.VMEM((CHUNK, CH), jnp.float32),
          pltpu.VMEM_SHARED((ACC_ROWS, CH), jnp.float32),
          pltpu.SemaphoreType.DMA,
          pltpu.SemaphoreType.DMA,
          pltpu.SemaphoreType.DMA,
          pltpu.SemaphoreType.DMA,
      ],
      compiler_params=pltpu.CompilerParams(needs_layout_passes=False),
  )(g_flat, row2, col2, zeros128)


def _full(shape):
  return pl.BlockSpec(shape, lambda i: (0,) * len(shape))


def _tca1_body(x_ref, w1a_ref, b1a_ref, w1b_ref, b1b_ref,
               w1c_ref, b1c_ref, h1_ref):
  x = x_ref[...]
  p = x @ w1a_ref[...] + b1a_ref[...]
  q = jax.nn.sigmoid(x @ w1b_ref[...] + b1b_ref[...])
  r = x @ w1c_ref[...] + b1c_ref[...]
  h1_ref[...] = jax.nn.relu(p * q + r)


def _tc_a1(x, w1a, b1a, w1b, b1b, w1c, b1c):
  return pl.pallas_call(
      _tca1_body,
      grid=(GRID,),
      in_specs=[
          pl.BlockSpec((BLK, C), lambda i: (i, 0)),
          _full((C, C)), _full((1, C)),
          _full((C, C)), _full((1, C)),
          _full((C, C)), _full((1, C)),
      ],
      out_specs=pl.BlockSpec((BLK, C), lambda i: (i, 0)),
      out_shape=jax.ShapeDtypeStruct((N, C), jnp.float32),
  )(x, w1a, b1a, w1b, b1b, w1c, b1c)


def _tca2_body(h1_ref, degs_ref, g2_ref, dis_ref):
  h1 = h1_ref[...]
  d = lax.dot_general(degs_ref[...], jnp.ones((NW, 1), jnp.float32),
                      (((1,), (0,)), ((), ())))
  dis = jnp.where(d > 0, lax.rsqrt(jnp.where(d > 0, d, 1.0)), 0.0)
  dis_ref[...] = dis
  g = dis * h1
  g2_ref[0, :, :] = g[:, :CH]
  g2_ref[1, :, :] = g[:, CH:]


def _tc_a2(h1, degs):
  return pl.pallas_call(
      _tca2_body,
      grid=(GRID2,),
      in_specs=[
          pl.BlockSpec((BLK2, C), lambda i: (i, 0)),
          pl.BlockSpec((BLK2, NW), lambda i: (i, 0)),
      ],
      out_specs=[
          pl.BlockSpec((NC, BLK2, CH), lambda i: (0, i, 0)),
          pl.BlockSpec((BLK2, 1), lambda i: (i, 0)),
      ],
      out_shape=[
          jax.ShapeDtypeStruct((NC, N, CH), jnp.float32),
          jax.ShapeDtypeStruct((N, 1), jnp.float32),
      ],
  )(h1, degs)


def _tcb1_body(h1_ref, th0_ref, cb_ref, u_ref):
  u_ref[...] = h1_ref[...] @ th0_ref[...] + cb_ref[...]


def _tc_b1(h1, th0, cb):
  return pl.pallas_call(
      _tcb1_body,
      grid=(GRID,),
      in_specs=[
          pl.BlockSpec((BLK, C), lambda i: (i, 0)),
          _full((C, C)), _full((1, C)),
      ],
      out_specs=pl.BlockSpec((BLK, C), lambda i: (i, 0)),
      out_shape=jax.ShapeDtypeStruct((N, C), jnp.float32),
  )(h1, th0, cb)


def _tcb2_body(u_ref, t2_ref, dis_ref, th1_ref,
               w2a_ref, b2a_ref, w2b_ref, b2b_ref, w2c_ref, b2c_ref,
               gamma_ref, beta_ref, wl_ref, bl_ref, out_ref):
  dis = dis_ref[...]
  t = jnp.concatenate([t2_ref[0, :, :], t2_ref[1, :, :]], axis=1)
  tx1 = (-dis) * t
  h2 = jax.nn.relu(u_ref[...] + tx1 @ th1_ref[...])

  p = h2 @ w2a_ref[...] + b2a_ref[...]
  q = jax.nn.sigmoid(h2 @ w2b_ref[...] + b2b_ref[...])
  r = h2 @ w2c_ref[...] + b2c_ref[...]
  h3 = jax.nn.relu(p * q + r)

  mean = jnp.mean(h3, axis=1, keepdims=True)
  var = jnp.mean((h3 - mean) ** 2, axis=1, keepdims=True)
  hn = (h3 - mean) / jnp.sqrt(var + 1e-5) * gamma_ref[...] + beta_ref[...]
  h4 = jax.nn.relu(hn)
  out_ref[...] = (jnp.sum(h4 * wl_ref[...], axis=1, keepdims=True)
                  + bl_ref[...])


def _tc_b2(u, t2, dis, th1, w2a, b2a, w2b, b2b, w2c, b2c,
           gamma, beta, wl, bl):
  return pl.pallas_call(
      _tcb2_body,
      grid=(GRID,),
      in_specs=[
          pl.BlockSpec((BLK, C), lambda i: (i, 0)),
          pl.BlockSpec((NC, BLK, CH), lambda i: (0, i, 0)),
          pl.BlockSpec((BLK, 1), lambda i: (i, 0)),
          _full((C, C)),
          _full((C, C)), _full((1, C)),
          _full((C, C)), _full((1, C)),
          _full((C, C)), _full((1, C)),
          pl.BlockSpec((BLK, 1), lambda i: (i, 0)),
          pl.BlockSpec((BLK, 1), lambda i: (i, 0)),
          _full((1, C)), _full((1, 1)),
      ],
      out_specs=pl.BlockSpec((BLK, 1), lambda i: (i, 0)),
      out_shape=jax.ShapeDtypeStruct((N, 1), jnp.float32),
  )(u, t2, dis, th1, w2a, b2a, w2b, b2b, w2c, b2c, gamma, beta, wl, bl)


def kernel(x, edge_index, W1a, b1a, W1b, b1b, W1c, b1c, th0, th1, cb,
           W2a, b2a, W2b, b2b, W2c, b2c, gamma, beta, Wl, bl):
  row = edge_index[0]
  col = edge_index[1]

  row1 = jnp.concatenate(
      [row, jnp.full((NW * EPT1 - E,), DEG_DUMMY, jnp.int32)]
  ).reshape(NW, EPT1)
  degs = _sc_degree(row1)

  h1 = _tc_a1(x, W1a, b1a[None, :], W1b, b1b[None, :], W1c, b1c[None, :])
  g2, dis = _tc_a2(h1, degs.T)

  rowp = jnp.concatenate(
      [row, jnp.zeros((NS * EPT2 - E,), jnp.int32)]
  ).reshape(NS, NWIN, WCH, CHUNK)
  colp = jnp.concatenate(
      [col, jnp.full((NS * EPT2 - E,), COL_DUMMY, jnp.int32)]
  ).reshape(NS, NWIN, WCH, CHUNK)
  row2 = jnp.concatenate([rowp[None], rowp[None] + N], axis=0)
  row2 = row2.reshape(NW * NWIN, WCH, CHUNK)
  col2 = jnp.concatenate([colp[None], colp[None]], axis=0)
  col2 = col2.reshape(NW * NWIN, WCH, CHUNK)
  zeros128 = jnp.zeros((CHUNK, CH), jnp.float32)

  t_flat = _sc_edge_agg(g2.reshape(NC * N, CH), row2, col2, zeros128)
  t2 = t_flat.reshape(NC, ACC_ROWS, CH)

  u = _tc_b1(h1, th0, cb[None, :])
  return _tc_b2(u, t2, dis, th1,
                W2a, b2a[None, :], W2b, b2b[None, :], W2c, b2c[None, :],
                gamma[:, None], beta[:, None], Wl.reshape(1, C),
                bl.reshape(1, 1))
